# baseline (device time: 256392 ns/iter reference)
import jax
import jax.numpy as jnp
from jax import lax
from jax.experimental import pallas as pl
from jax.experimental.pallas import tpu as pltpu

M = 4096
N = 4096
KS = 2048
HM = M // 2
NC = 256
N_CHUNKS = N // NC
SLOTS = 3


def kernel(A, B):
    def body(
        a_hbm, b_hbm, out_hbm,
        af_buf, a16_buf, bf_buf, b16_buf,
        pc16_buf, px16_buf, rcf_buf, rc16_buf, ry16_buf, ost_buf,
        a_sem, b_sems, o0_sems, o1_sems,
        x_send, x_recv, y_send, y_recv,
        x_credit, y_credit,
    ):
        my_x = lax.axis_index("x")
        my_y = lax.axis_index("y")
        x_nbr = (1 - my_x, my_y)
        y_nbr = (my_x, 1 - my_y)

        a_cp = pltpu.make_async_copy(
            a_hbm.at[pl.ds(my_y * HM, HM), :], af_buf, a_sem
        )
        a_cp.start()

        def b_load(c):
            return pltpu.make_async_copy(
                b_hbm.at[:, pl.ds(c * NC, NC)],
                bf_buf.at[c % SLOTS],
                b_sems.at[c % SLOTS],
            )

        b_cps = {}
        for c in range(min(SLOTS, N_CHUNKS)):
            b_cps[c] = b_load(c)
            b_cps[c].start()

        barrier_sem = pltpu.get_barrier_semaphore()
        for nbr in (x_nbr, y_nbr):
            pl.semaphore_signal(
                barrier_sem, inc=1,
                device_id=nbr, device_id_type=pl.DeviceIdType.MESH,
            )
        pl.semaphore_wait(barrier_sem, 2)

        a_cp.wait()
        a16_buf[:, :] = af_buf[:, :].astype(jnp.bfloat16)

        def x_desc(c):
            s = c % SLOTS
            return pltpu.make_async_remote_copy(
                src_ref=pc16_buf.at[s], dst_ref=px16_buf.at[s],
                send_sem=x_send.at[s], recv_sem=x_recv.at[s],
                device_id=x_nbr, device_id_type=pl.DeviceIdType.MESH,
            )

        def y_desc(c):
            s = c % SLOTS
            return pltpu.make_async_remote_copy(
                src_ref=rc16_buf.at[s], dst_ref=ry16_buf.at[s],
                send_sem=y_send.at[s], recv_sem=y_recv.at[s],
                device_id=y_nbr, device_id_type=pl.DeviceIdType.MESH,
            )

        x_rdmas = {}
        y_rdmas = {}
        o0_cps = {}
        o1_cps = {}
        for it in range(N_CHUNKS + 2):
            c = it
            if c < N_CHUNKS:
                s = c % SLOTS
                if c >= SLOTS:
                    x_rdmas[c - SLOTS].wait_send()
                b_cps[c].wait()
                b16_buf[s, :, :] = bf_buf[s].astype(jnp.bfloat16)
                pc16_buf[s, :, :] = jnp.dot(
                    a16_buf[:, :], b16_buf[s],
                    preferred_element_type=jnp.float32,
                ).astype(jnp.bfloat16)
                if c + SLOTS < N_CHUNKS:
                    b_cps[c + SLOTS] = b_load(c + SLOTS)
                    b_cps[c + SLOTS].start()
                if c >= SLOTS:
                    pl.semaphore_wait(x_credit, 1)
                x_rdmas[c] = x_desc(c)
                x_rdmas[c].start()

            d = it - 1
            if 0 <= d < N_CHUNKS:
                sd = d % SLOTS
                x_rdmas[d].wait_recv()
                if d >= SLOTS:
                    y_rdmas[d - SLOTS].wait_send()
                if d >= 2:
                    o0_cps[d - 2].wait()
                rsum = (
                    pc16_buf[sd].astype(jnp.float32)
                    + px16_buf[sd].astype(jnp.float32)
                )
                rcf_buf[d % 2, :, :] = rsum
                rc16_buf[sd, :, :] = rsum.astype(jnp.bfloat16)
                if d + SLOTS < N_CHUNKS:
                    pl.semaphore_signal(
                        x_credit, inc=1,
                        device_id=x_nbr, device_id_type=pl.DeviceIdType.MESH,
                    )
                if d >= SLOTS:
                    pl.semaphore_wait(y_credit, 1)
                y_rdmas[d] = y_desc(d)
                y_rdmas[d].start()

            e = it - 2
            if 0 <= e < N_CHUNKS:
                se = e % SLOTS
                y_rdmas[e].wait_recv()
                if e >= 2:
                    o1_cps[e - 2].wait()
                ost_buf[e % 2, :, :] = ry16_buf[se].astype(jnp.float32)
                if e + SLOTS < N_CHUNKS:
                    pl.semaphore_signal(
                        y_credit, inc=1,
                        device_id=y_nbr, device_id_type=pl.DeviceIdType.MESH,
                    )
                o0_cps[e] = pltpu.make_async_copy(
                    rcf_buf.at[e % 2],
                    out_hbm.at[pl.ds(my_y * HM, HM), pl.ds(e * NC, NC)],
                    o0_sems.at[e % 2],
                )
                o1_cps[e] = pltpu.make_async_copy(
                    ost_buf.at[e % 2],
                    out_hbm.at[pl.ds((1 - my_y) * HM, HM), pl.ds(e * NC, NC)],
                    o1_sems.at[e % 2],
                )
                o0_cps[e].start()
                o1_cps[e].start()

        for c in range(max(0, N_CHUNKS - SLOTS), N_CHUNKS):
            x_rdmas[c].wait_send()
            y_rdmas[c].wait_send()
        for c in (N_CHUNKS - 2, N_CHUNKS - 1):
            o0_cps[c].wait()
            o1_cps[c].wait()

    return pl.pallas_call(
        body,
        out_shape=jax.ShapeDtypeStruct((M, N), jnp.float32),
        in_specs=[
            pl.BlockSpec(memory_space=pl.ANY),
            pl.BlockSpec(memory_space=pl.ANY),
        ],
        out_specs=pl.BlockSpec(memory_space=pl.ANY),
        scratch_shapes=[
            pltpu.VMEM((HM, KS), jnp.float32),
            pltpu.VMEM((HM, KS), jnp.bfloat16),
            pltpu.VMEM((SLOTS, KS, NC), jnp.float32),
            pltpu.VMEM((SLOTS, KS, NC), jnp.bfloat16),
            pltpu.VMEM((SLOTS, HM, NC), jnp.bfloat16),
            pltpu.VMEM((SLOTS, HM, NC), jnp.bfloat16),
            pltpu.VMEM((2, HM, NC), jnp.float32),
            pltpu.VMEM((SLOTS, HM, NC), jnp.bfloat16),
            pltpu.VMEM((SLOTS, HM, NC), jnp.bfloat16),
            pltpu.VMEM((2, HM, NC), jnp.float32),
            pltpu.SemaphoreType.DMA,
            pltpu.SemaphoreType.DMA((SLOTS,)),
            pltpu.SemaphoreType.DMA((2,)),
            pltpu.SemaphoreType.DMA((2,)),
            pltpu.SemaphoreType.DMA((SLOTS,)),
            pltpu.SemaphoreType.DMA((SLOTS,)),
            pltpu.SemaphoreType.DMA((SLOTS,)),
            pltpu.SemaphoreType.DMA((SLOTS,)),
            pltpu.SemaphoreType.REGULAR,
            pltpu.SemaphoreType.REGULAR,
        ],
        compiler_params=pltpu.CompilerParams(
            collective_id=0,
            vmem_limit_bytes=63 * 1024 * 1024,
        ),
    )(A, B)


# device time: 255625 ns/iter; 1.0030x vs baseline; 1.0030x over previous
import jax
import jax.numpy as jnp
from jax import lax
from jax.experimental import pallas as pl
from jax.experimental.pallas import tpu as pltpu

M = 4096
N = 4096
KS = 2048
HM = M // 2
NC = 256
N_CHUNKS = N // NC
SLOTS = 4
B_SLOTS = 3
AQ = HM // 4


def kernel(A, B):
    def body(
        a_hbm, b_hbm, out_hbm,
        af_buf, a16_buf, bf_buf, b16_buf,
        pc16_buf, px16_buf, rcf_buf, rc16_buf, ry16_buf, ost_buf,
        a_sems, b_sems, o0_sems, o1_sems,
        x_send, x_recv, y_send, y_recv,
        x_credit, y_credit,
    ):
        my_x = lax.axis_index("x")
        my_y = lax.axis_index("y")
        x_nbr = (1 - my_x, my_y)
        y_nbr = (my_x, 1 - my_y)

        def a_load(q):
            return pltpu.make_async_copy(
                a_hbm.at[pl.ds(my_y * HM + q * AQ, AQ), :],
                af_buf.at[q % 2],
                a_sems.at[q % 2],
            )

        a_cps = {0: a_load(0)}
        a_cps[0].start()
        a_cps[1] = a_load(1)
        a_cps[1].start()

        def b_load(c):
            return pltpu.make_async_copy(
                b_hbm.at[:, pl.ds(c * NC, NC)],
                bf_buf.at[c % B_SLOTS],
                b_sems.at[c % B_SLOTS],
            )

        b_cps = {}
        for c in range(min(B_SLOTS, N_CHUNKS)):
            b_cps[c] = b_load(c)
            b_cps[c].start()

        barrier_sem = pltpu.get_barrier_semaphore()
        for nbr in (x_nbr, y_nbr):
            pl.semaphore_signal(
                barrier_sem, inc=1,
                device_id=nbr, device_id_type=pl.DeviceIdType.MESH,
            )
        pl.semaphore_wait(barrier_sem, 2)

        for q in range(4):
            a_cps[q].wait()
            a16_buf[pl.ds(q * AQ, AQ), :] = af_buf[q % 2].astype(jnp.bfloat16)
            if q + 2 < 4:
                a_cps[q + 2] = a_load(q + 2)
                a_cps[q + 2].start()

        def x_desc(c):
            s = c % SLOTS
            return pltpu.make_async_remote_copy(
                src_ref=pc16_buf.at[s], dst_ref=px16_buf.at[s],
                send_sem=x_send.at[s], recv_sem=x_recv.at[s],
                device_id=x_nbr, device_id_type=pl.DeviceIdType.MESH,
            )

        def y_desc(c):
            s = c % SLOTS
            return pltpu.make_async_remote_copy(
                src_ref=rc16_buf.at[s], dst_ref=ry16_buf.at[s],
                send_sem=y_send.at[s], recv_sem=y_recv.at[s],
                device_id=y_nbr, device_id_type=pl.DeviceIdType.MESH,
            )

        x_rdmas = {}
        y_rdmas = {}
        o0_cps = {}
        o1_cps = {}
        for it in range(N_CHUNKS + 4):
            c = it
            if c < N_CHUNKS:
                s = c % SLOTS
                if c >= SLOTS:
                    x_rdmas[c - SLOTS].wait_send()
                b_cps[c].wait()
                b16_buf[c % 2, :, :] = bf_buf[c % B_SLOTS].astype(jnp.bfloat16)
                pc16_buf[s, :, :] = jnp.dot(
                    a16_buf[:, :], b16_buf[c % 2],
                    preferred_element_type=jnp.float32,
                ).astype(jnp.bfloat16)
                if c + B_SLOTS < N_CHUNKS:
                    b_cps[c + B_SLOTS] = b_load(c + B_SLOTS)
                    b_cps[c + B_SLOTS].start()
                if c >= SLOTS:
                    pl.semaphore_wait(x_credit, 1)
                x_rdmas[c] = x_desc(c)
                x_rdmas[c].start()

            d = it - 2
            if 0 <= d < N_CHUNKS:
                sd = d % SLOTS
                x_rdmas[d].wait_recv()
                if d >= SLOTS:
                    y_rdmas[d - SLOTS].wait_send()
                    o0_cps[d - SLOTS].wait()
                rsum = (
                    pc16_buf[sd].astype(jnp.float32)
                    + px16_buf[sd].astype(jnp.float32)
                )
                rcf_buf[d % SLOTS, :, :] = rsum
                rc16_buf[sd, :, :] = rsum.astype(jnp.bfloat16)
                if d + SLOTS < N_CHUNKS:
                    pl.semaphore_signal(
                        x_credit, inc=1,
                        device_id=x_nbr, device_id_type=pl.DeviceIdType.MESH,
                    )
                if d >= SLOTS:
                    pl.semaphore_wait(y_credit, 1)
                y_rdmas[d] = y_desc(d)
                y_rdmas[d].start()

            e = it - 4
            if 0 <= e < N_CHUNKS:
                se = e % SLOTS
                y_rdmas[e].wait_recv()
                if e >= 2:
                    o1_cps[e - 2].wait()
                ost_buf[e % 2, :, :] = ry16_buf[se].astype(jnp.float32)
                if e + SLOTS < N_CHUNKS:
                    pl.semaphore_signal(
                        y_credit, inc=1,
                        device_id=y_nbr, device_id_type=pl.DeviceIdType.MESH,
                    )
                o0_cps[e] = pltpu.make_async_copy(
                    rcf_buf.at[e % SLOTS],
                    out_hbm.at[pl.ds(my_y * HM, HM), pl.ds(e * NC, NC)],
                    o0_sems.at[e % 2],
                )
                o1_cps[e] = pltpu.make_async_copy(
                    ost_buf.at[e % 2],
                    out_hbm.at[pl.ds((1 - my_y) * HM, HM), pl.ds(e * NC, NC)],
                    o1_sems.at[e % 2],
                )
                o0_cps[e].start()
                o1_cps[e].start()

        for c in range(max(0, N_CHUNKS - SLOTS), N_CHUNKS):
            x_rdmas[c].wait_send()
            y_rdmas[c].wait_send()
            o0_cps[c].wait()
        for c in (N_CHUNKS - 2, N_CHUNKS - 1):
            o1_cps[c].wait()

    return pl.pallas_call(
        body,
        out_shape=jax.ShapeDtypeStruct((M, N), jnp.float32),
        in_specs=[
            pl.BlockSpec(memory_space=pl.ANY),
            pl.BlockSpec(memory_space=pl.ANY),
        ],
        out_specs=pl.BlockSpec(memory_space=pl.ANY),
        scratch_shapes=[
            pltpu.VMEM((2, AQ, KS), jnp.float32),
            pltpu.VMEM((HM, KS), jnp.bfloat16),
            pltpu.VMEM((B_SLOTS, KS, NC), jnp.float32),
            pltpu.VMEM((2, KS, NC), jnp.bfloat16),
            pltpu.VMEM((SLOTS, HM, NC), jnp.bfloat16),
            pltpu.VMEM((SLOTS, HM, NC), jnp.bfloat16),
            pltpu.VMEM((SLOTS, HM, NC), jnp.float32),
            pltpu.VMEM((SLOTS, HM, NC), jnp.bfloat16),
            pltpu.VMEM((SLOTS, HM, NC), jnp.bfloat16),
            pltpu.VMEM((2, HM, NC), jnp.float32),
            pltpu.SemaphoreType.DMA((2,)),
            pltpu.SemaphoreType.DMA((B_SLOTS,)),
            pltpu.SemaphoreType.DMA((2,)),
            pltpu.SemaphoreType.DMA((2,)),
            pltpu.SemaphoreType.DMA((SLOTS,)),
            pltpu.SemaphoreType.DMA((SLOTS,)),
            pltpu.SemaphoreType.DMA((SLOTS,)),
            pltpu.SemaphoreType.DMA((SLOTS,)),
            pltpu.SemaphoreType.REGULAR,
            pltpu.SemaphoreType.REGULAR,
        ],
        compiler_params=pltpu.CompilerParams(
            collective_id=0,
            vmem_limit_bytes=63 * 1024 * 1024,
        ),
    )(A, B)
